# baseline (speedup 1.0000x reference)
import jax
import jax.numpy as jnp
from jax.experimental import pallas as pl
from jax.experimental.pallas import tpu as pltpu

_NBLK = 4


def _gcn_body(A_hbm, xT_ref, W1_ref, b1_ref, W2_ref, b2_ref, oT_ref,
              A_vmem, Abf_ref, sems):
    n = A_vmem.shape[0]
    bk = n // _NBLK
    copies = [
        pltpu.make_async_copy(
            A_hbm.at[pl.ds(i * bk, bk), :],
            A_vmem.at[pl.ds(i * bk, bk), :],
            sems.at[i],
        )
        for i in range(_NBLK)
    ]
    for c in copies:
        c.start()

    xT = xT_ref[...]
    h1 = jnp.dot(W1_ref[...].T, xT, preferred_element_type=jnp.float32)

    cs = jnp.zeros((1, n), jnp.float32)
    for i, c in enumerate(copies):
        c.wait()
        blk = A_vmem[pl.ds(i * bk, bk), :]
        cs = cs + jnp.sum(blk, axis=0, keepdims=True)
        row = jax.lax.broadcasted_iota(jnp.int32, (bk, n), 0) + i * bk
        col = jax.lax.broadcasted_iota(jnp.int32, (bk, n), 1)
        eye = jnp.where(row == col, 1.0, 0.0)
        Abf_ref[pl.ds(i * bk, bk), :] = (blk + eye).astype(jnp.bfloat16)

    A = Abf_ref[...]
    dis = jax.lax.rsqrt(cs + 1.0)
    g1 = h1 * dis
    t1 = jnp.dot(g1.astype(jnp.bfloat16), A,
                 preferred_element_type=jnp.float32)
    o1 = jnp.maximum(t1 * dis + b1_ref[...].T, 0.0)
    h2 = jnp.dot(W2_ref[...].T, o1, preferred_element_type=jnp.float32)
    g2 = h2 * dis
    t2 = jnp.dot(g2.astype(jnp.bfloat16), A,
                 preferred_element_type=jnp.float32)
    oT_ref[...] = t2 * dis + b2_ref[...].T + xT


def kernel(A, x, W1, b1, W2, b2):
    n, f = x.shape
    out_t = pl.pallas_call(
        _gcn_body,
        in_specs=[
            pl.BlockSpec(memory_space=pl.ANY),
            pl.BlockSpec((f, n), lambda: (0, 0)),
            pl.BlockSpec((f, f), lambda: (0, 0)),
            pl.BlockSpec((1, f), lambda: (0, 0)),
            pl.BlockSpec((f, f), lambda: (0, 0)),
            pl.BlockSpec((1, f), lambda: (0, 0)),
        ],
        out_specs=pl.BlockSpec((f, n), lambda: (0, 0)),
        out_shape=jax.ShapeDtypeStruct((f, n), jnp.float32),
        scratch_shapes=[
            pltpu.VMEM((n, n), jnp.float32),
            pltpu.VMEM((n, n), jnp.bfloat16),
            pltpu.SemaphoreType.DMA((_NBLK,)),
        ],
    )(A, x.T, W1, b1.reshape(1, f), W2, b2.reshape(1, f))
    return out_t.T.astype(jnp.float64)


# speedup vs baseline: 1.0055x; 1.0055x over previous
import jax
import jax.numpy as jnp
from jax.experimental import pallas as pl
from jax.experimental.pallas import tpu as pltpu

_NBLK = 8


def _gcn_body(A_hbm, xT_ref, W1_ref, b1_ref, W2_ref, b2_ref, oT_ref,
              A_vmem, Abf_ref, sems):
    n = A_vmem.shape[0]
    bk = n // _NBLK
    copies = [
        pltpu.make_async_copy(
            A_hbm.at[pl.ds(i * bk, bk), :],
            A_vmem.at[pl.ds(i * bk, bk), :],
            sems.at[i],
        )
        for i in range(_NBLK)
    ]
    for c in copies:
        c.start()

    xT = xT_ref[...]
    h1 = jnp.dot(W1_ref[...].T, xT, preferred_element_type=jnp.float32)

    cs = jnp.zeros((1, n), jnp.float32)
    for i, c in enumerate(copies):
        c.wait()
        blk = A_vmem[pl.ds(i * bk, bk), :]
        cs = cs + jnp.sum(blk, axis=0, keepdims=True)
        row = jax.lax.broadcasted_iota(jnp.int32, (bk, n), 0) + i * bk
        col = jax.lax.broadcasted_iota(jnp.int32, (bk, n), 1)
        eye = jnp.where(row == col, 1.0, 0.0)
        Abf_ref[pl.ds(i * bk, bk), :] = (blk + eye).astype(jnp.bfloat16)

    A = Abf_ref[...]
    dis = jax.lax.rsqrt(cs + 1.0)
    g1 = h1 * dis
    t1 = jnp.dot(g1.astype(jnp.bfloat16), A,
                 preferred_element_type=jnp.float32)
    o1 = jnp.maximum(t1 * dis + b1_ref[...].T, 0.0)
    h2 = jnp.dot(W2_ref[...].T, o1, preferred_element_type=jnp.float32)
    g2 = h2 * dis
    t2 = jnp.dot(g2.astype(jnp.bfloat16), A,
                 preferred_element_type=jnp.float32)
    oT_ref[...] = t2 * dis + b2_ref[...].T + xT


def kernel(A, x, W1, b1, W2, b2):
    n, f = x.shape
    out_t = pl.pallas_call(
        _gcn_body,
        in_specs=[
            pl.BlockSpec(memory_space=pl.ANY),
            pl.BlockSpec((f, n), lambda: (0, 0)),
            pl.BlockSpec((f, f), lambda: (0, 0)),
            pl.BlockSpec((1, f), lambda: (0, 0)),
            pl.BlockSpec((f, f), lambda: (0, 0)),
            pl.BlockSpec((1, f), lambda: (0, 0)),
        ],
        out_specs=pl.BlockSpec((f, n), lambda: (0, 0)),
        out_shape=jax.ShapeDtypeStruct((f, n), jnp.float32),
        scratch_shapes=[
            pltpu.VMEM((n, n), jnp.float32),
            pltpu.VMEM((n, n), jnp.bfloat16),
            pltpu.SemaphoreType.DMA((_NBLK,)),
        ],
    )(A, x.T, W1, b1.reshape(1, f), W2, b2.reshape(1, f))
    return out_t.T.astype(jnp.float64)


# speedup vs baseline: 1.0114x; 1.0059x over previous
import jax
import jax.numpy as jnp
from jax.experimental import pallas as pl
from jax.experimental.pallas import tpu as pltpu

_CHUNKS = (512, 512, 256, 256, 192, 128, 128, 64)


def _gcn_body(A_hbm, xT_ref, W1_ref, b1_ref, W2_ref, b2_ref, oT_ref,
              A_vmem, Abf_ref, sems):
    n = A_vmem.shape[0]
    offs = [0]
    for c in _CHUNKS:
        offs.append(offs[-1] + c)
    copies = [
        pltpu.make_async_copy(
            A_hbm.at[pl.ds(offs[i], _CHUNKS[i]), :],
            A_vmem.at[pl.ds(offs[i], _CHUNKS[i]), :],
            sems.at[i],
        )
        for i in range(len(_CHUNKS))
    ]
    for c in copies:
        c.start()

    xT = xT_ref[...]
    h1 = jnp.dot(W1_ref[...].T, xT, preferred_element_type=jnp.float32)

    cs = jnp.zeros((1, n), jnp.float32)
    for i, c in enumerate(copies):
        c.wait()
        bk = _CHUNKS[i]
        blk = A_vmem[pl.ds(offs[i], bk), :]
        cs = cs + jnp.sum(blk, axis=0, keepdims=True)
        row = jax.lax.broadcasted_iota(jnp.int32, (bk, n), 0) + offs[i]
        col = jax.lax.broadcasted_iota(jnp.int32, (bk, n), 1)
        eye = jnp.where(row == col, 1.0, 0.0)
        Abf_ref[pl.ds(offs[i], bk), :] = (blk + eye).astype(jnp.bfloat16)

    A = Abf_ref[...]
    dis = jax.lax.rsqrt(cs + 1.0)
    g1 = h1 * dis
    t1 = jnp.dot(g1.astype(jnp.bfloat16), A,
                 preferred_element_type=jnp.float32)
    o1 = jnp.maximum(t1 * dis + b1_ref[...].T, 0.0)
    h2 = jnp.dot(W2_ref[...].T, o1, preferred_element_type=jnp.float32)
    g2 = h2 * dis
    t2 = jnp.dot(g2.astype(jnp.bfloat16), A,
                 preferred_element_type=jnp.float32)
    oT_ref[...] = t2 * dis + b2_ref[...].T + xT


def kernel(A, x, W1, b1, W2, b2):
    n, f = x.shape
    out_t = pl.pallas_call(
        _gcn_body,
        in_specs=[
            pl.BlockSpec(memory_space=pl.ANY),
            pl.BlockSpec((f, n), lambda: (0, 0)),
            pl.BlockSpec((f, f), lambda: (0, 0)),
            pl.BlockSpec((1, f), lambda: (0, 0)),
            pl.BlockSpec((f, f), lambda: (0, 0)),
            pl.BlockSpec((1, f), lambda: (0, 0)),
        ],
        out_specs=pl.BlockSpec((f, n), lambda: (0, 0)),
        out_shape=jax.ShapeDtypeStruct((f, n), jnp.float32),
        scratch_shapes=[
            pltpu.VMEM((n, n), jnp.float32),
            pltpu.VMEM((n, n), jnp.bfloat16),
            pltpu.SemaphoreType.DMA((len(_CHUNKS),)),
        ],
    )(A, x.T, W1, b1.reshape(1, f), W2, b2.reshape(1, f))
    return out_t.T.astype(jnp.float64)
